# baseline (speedup 1.0000x reference)
import functools

import numpy as np

import jax
import jax.numpy as jnp
from jax import lax
from jax.experimental import pallas as pl
from jax.experimental.pallas import tpu as pltpu
from jax.experimental.shard_map import shard_map
from jax.sharding import Mesh, PartitionSpec as P

_K = 16
_EPS = 1e-5
_INF = 3.0e38


def _knn_agg_body(coords_ref, coordsT_ref, cb_ref, ctb_ref, fhi_ref, flo_ref,
                  out_ref, d2_ref, *, rows, cols, k):
    c = coords_ref[0]
    ct = coordsT_ref[0]
    cb = cb_ref[0].astype(jnp.float32)
    ctb = ctb_ref[0].astype(jnp.float32)

    dot = (cb[:, 0:1] * ctb[0:1, :] + cb[:, 1:2] * ctb[1:2, :]
           + cb[:, 2:3] * ctb[2:3, :])
    rowsq = c[:, 0:1] * c[:, 0:1] + c[:, 1:2] * c[:, 1:2] + c[:, 2:3] * c[:, 2:3]
    colsq = ct[0:1, :] * ct[0:1, :] + ct[1:2, :] * ct[1:2, :] + ct[2:3, :] * ct[2:3, :]
    d2 = jnp.maximum((rowsq + colsq) - 2.0 * dot, 0.0)
    d2_ref[...] = jnp.sqrt(d2)

    iota = lax.broadcasted_iota(jnp.int32, (rows, cols), 1)

    def body(_, carry):
        del carry
        d = d2_ref[...]
        j = jnp.argmin(d, axis=1)[:, None]
        d2_ref[...] = jnp.where(iota == j, _INF, d)
        return 0

    lax.fori_loop(0, k, body, 0)

    adjb = (d2_ref[...] == _INF).astype(jnp.bfloat16)
    agg = (jnp.dot(adjb, fhi_ref[0], preferred_element_type=jnp.float32)
           + jnp.dot(adjb, flo_ref[0], preferred_element_type=jnp.float32))
    out_ref[0] = agg * (1.0 / k)


def _mlp_chain_body(agg_ref, w1t_ref, b1_ref, g1_ref, be1_ref,
                    wat_ref, ba_ref, ga_ref, bea_ref,
                    w2t_ref, b2_ref, g2_ref, be2_ref, out_ref):
    def bn_relu(y, g, be):
        m = jnp.mean(y, axis=0, keepdims=True)
        v = jnp.mean((y - m) * (y - m), axis=0, keepdims=True)
        return jax.nn.relu((y - m) / jnp.sqrt(v + _EPS) * g + be)

    def mm(x, wt_ref, b_ref):
        return jnp.dot(x.astype(jnp.bfloat16), wt_ref[...],
                       preferred_element_type=jnp.float32) + b_ref[...]

    a = bn_relu(mm(agg_ref[...], w1t_ref, b1_ref), g1_ref[...], be1_ref[...])
    attn = jax.nn.sigmoid(bn_relu(mm(a, wat_ref, ba_ref),
                                  ga_ref[...], bea_ref[...]))
    f = a * attn + a
    out_ref[...] = bn_relu(mm(f, w2t_ref, b2_ref), g2_ref[...], be2_ref[...])


def kernel(coords, feats, W1, b1, g1, be1, Wa, ba, ga, bea, W2, b2, g2, be2):
    B, N, _ = coords.shape
    IN_CH = feats.shape[-1]
    MID = W1.shape[0]
    OUT_CH = W2.shape[0]
    rows = 256 if N % 256 == 0 else N

    coordsT = jnp.swapaxes(coords, 1, 2)
    cb = coords.astype(jnp.bfloat16)
    ctb = coordsT.astype(jnp.bfloat16)
    fhi = feats.astype(jnp.bfloat16)
    flo = (feats - fhi.astype(jnp.float32)).astype(jnp.bfloat16)

    def knn_call(c, ct, cbl, ctbl, fh, fl):
        bl = c.shape[0]
        return pl.pallas_call(
            functools.partial(_knn_agg_body, rows=rows, cols=N, k=_K),
            grid=(bl, N // rows),
            in_specs=[
                pl.BlockSpec((1, rows, 3), lambda b, r: (b, r, 0)),
                pl.BlockSpec((1, 3, N), lambda b, r: (b, 0, 0)),
                pl.BlockSpec((1, rows, 3), lambda b, r: (b, r, 0)),
                pl.BlockSpec((1, 3, N), lambda b, r: (b, 0, 0)),
                pl.BlockSpec((1, N, IN_CH), lambda b, r: (b, 0, 0)),
                pl.BlockSpec((1, N, IN_CH), lambda b, r: (b, 0, 0)),
            ],
            out_specs=pl.BlockSpec((1, rows, IN_CH), lambda b, r: (b, r, 0)),
            out_shape=jax.ShapeDtypeStruct((bl, N, IN_CH), jnp.float32),
            scratch_shapes=[
                pltpu.VMEM((rows, N), jnp.float32),
            ],
        )(c, ct, cbl, ctbl, fh, fl)

    row2 = lambda v: v.reshape(1, -1)

    def mlp_call(agg2d, *ws):
        return pl.pallas_call(
            _mlp_chain_body,
            out_shape=jax.ShapeDtypeStruct((agg2d.shape[0], OUT_CH), jnp.float32),
        )(agg2d, *ws)

    weights = (W1.T.astype(jnp.bfloat16), row2(b1), row2(g1), row2(be1),
               Wa.T.astype(jnp.bfloat16), row2(ba), row2(ga), row2(bea),
               W2.T.astype(jnp.bfloat16), row2(b2), row2(g2), row2(be2))

    devs = jax.devices()
    nd = 2 if (len(devs) >= 2 and B % 2 == 0) else 1
    if nd > 1:
        mesh = Mesh(np.asarray(devs[:nd]), ("d",))
        agg = shard_map(
            knn_call, mesh=mesh,
            in_specs=(P("d"), P("d"), P("d"), P("d"), P("d"), P("d")),
            out_specs=P("d"), check_rep=False,
        )(coords, coordsT, cb, ctb, fhi, flo)
        out = shard_map(
            mlp_call, mesh=mesh,
            in_specs=(P(),) * 13,
            out_specs=P(), check_rep=False,
        )(agg.reshape(B * N, IN_CH), *weights)
    else:
        agg = knn_call(coords, coordsT, cb, ctb, fhi, flo)
        out = mlp_call(agg.reshape(B * N, IN_CH), *weights)

    return out.reshape(B, N, OUT_CH)


# speedup vs baseline: 1.0560x; 1.0560x over previous
import functools

import numpy as np

import jax
import jax.numpy as jnp
from jax import lax
from jax.experimental import pallas as pl
from jax.experimental.pallas import tpu as pltpu
from jax.experimental.shard_map import shard_map
from jax.sharding import Mesh, PartitionSpec as P

_K = 16
_EPS = 1e-5
_INF = 3.0e38


def _knn_agg_body(coords_ref, coordsT_ref, cb_ref, ctb_ref, fhi_ref, flo_ref,
                  out_ref, d2_ref, *, rows, cols, k):
    c = coords_ref[0]
    ct = coordsT_ref[0]
    cb = cb_ref[0].astype(jnp.float32)
    ctb = ctb_ref[0].astype(jnp.float32)

    dot = (cb[:, 0:1] * ctb[0:1, :] + cb[:, 1:2] * ctb[1:2, :]
           + cb[:, 2:3] * ctb[2:3, :])
    rowsq = c[:, 0:1] * c[:, 0:1] + c[:, 1:2] * c[:, 1:2] + c[:, 2:3] * c[:, 2:3]
    colsq = ct[0:1, :] * ct[0:1, :] + ct[1:2, :] * ct[1:2, :] + ct[2:3, :] * ct[2:3, :]
    d2 = jnp.maximum((rowsq + colsq) - 2.0 * dot, 0.0)
    d2_ref[...] = jnp.sqrt(d2)

    iota = lax.broadcasted_iota(jnp.int32, (rows, cols), 1)

    def body(_, carry):
        del carry
        d = d2_ref[...]
        g = jnp.min(d, axis=1, keepdims=True)
        cand = jnp.where(d == g, iota, cols)
        j = jnp.min(cand, axis=1, keepdims=True)
        d2_ref[...] = jnp.where(iota == j, _INF, d)
        return 0

    lax.fori_loop(0, k, body, 0)

    adjb = (d2_ref[...] == _INF).astype(jnp.bfloat16)
    agg = (jnp.dot(adjb, fhi_ref[0], preferred_element_type=jnp.float32)
           + jnp.dot(adjb, flo_ref[0], preferred_element_type=jnp.float32))
    out_ref[0] = agg * (1.0 / k)


def _mlp_chain_body(agg_ref, w1t_ref, b1_ref, g1_ref, be1_ref,
                    wat_ref, ba_ref, ga_ref, bea_ref,
                    w2t_ref, b2_ref, g2_ref, be2_ref, out_ref):
    def bn_relu(y, g, be):
        m = jnp.mean(y, axis=0, keepdims=True)
        v = jnp.mean((y - m) * (y - m), axis=0, keepdims=True)
        return jax.nn.relu((y - m) / jnp.sqrt(v + _EPS) * g + be)

    def mm(x, wt_ref, b_ref):
        return jnp.dot(x.astype(jnp.bfloat16), wt_ref[...],
                       preferred_element_type=jnp.float32) + b_ref[...]

    a = bn_relu(mm(agg_ref[...], w1t_ref, b1_ref), g1_ref[...], be1_ref[...])
    attn = jax.nn.sigmoid(bn_relu(mm(a, wat_ref, ba_ref),
                                  ga_ref[...], bea_ref[...]))
    f = a * attn + a
    out_ref[...] = bn_relu(mm(f, w2t_ref, b2_ref), g2_ref[...], be2_ref[...])


def kernel(coords, feats, W1, b1, g1, be1, Wa, ba, ga, bea, W2, b2, g2, be2):
    B, N, _ = coords.shape
    IN_CH = feats.shape[-1]
    MID = W1.shape[0]
    OUT_CH = W2.shape[0]
    rows = 256 if N % 256 == 0 else N

    coordsT = jnp.swapaxes(coords, 1, 2)
    cb = coords.astype(jnp.bfloat16)
    ctb = coordsT.astype(jnp.bfloat16)
    fhi = feats.astype(jnp.bfloat16)
    flo = (feats - fhi.astype(jnp.float32)).astype(jnp.bfloat16)

    def knn_call(c, ct, cbl, ctbl, fh, fl):
        bl = c.shape[0]
        return pl.pallas_call(
            functools.partial(_knn_agg_body, rows=rows, cols=N, k=_K),
            grid=(bl, N // rows),
            in_specs=[
                pl.BlockSpec((1, rows, 3), lambda b, r: (b, r, 0)),
                pl.BlockSpec((1, 3, N), lambda b, r: (b, 0, 0)),
                pl.BlockSpec((1, rows, 3), lambda b, r: (b, r, 0)),
                pl.BlockSpec((1, 3, N), lambda b, r: (b, 0, 0)),
                pl.BlockSpec((1, N, IN_CH), lambda b, r: (b, 0, 0)),
                pl.BlockSpec((1, N, IN_CH), lambda b, r: (b, 0, 0)),
            ],
            out_specs=pl.BlockSpec((1, rows, IN_CH), lambda b, r: (b, r, 0)),
            out_shape=jax.ShapeDtypeStruct((bl, N, IN_CH), jnp.float32),
            scratch_shapes=[
                pltpu.VMEM((rows, N), jnp.float32),
            ],
        )(c, ct, cbl, ctbl, fh, fl)

    row2 = lambda v: v.reshape(1, -1)

    def mlp_call(agg2d, *ws):
        return pl.pallas_call(
            _mlp_chain_body,
            out_shape=jax.ShapeDtypeStruct((agg2d.shape[0], OUT_CH), jnp.float32),
        )(agg2d, *ws)

    weights = (W1.T.astype(jnp.bfloat16), row2(b1), row2(g1), row2(be1),
               Wa.T.astype(jnp.bfloat16), row2(ba), row2(ga), row2(bea),
               W2.T.astype(jnp.bfloat16), row2(b2), row2(g2), row2(be2))

    devs = jax.devices()
    nd = 2 if (len(devs) >= 2 and B % 2 == 0) else 1
    if nd > 1:
        mesh = Mesh(np.asarray(devs[:nd]), ("d",))
        agg = shard_map(
            knn_call, mesh=mesh,
            in_specs=(P("d"), P("d"), P("d"), P("d"), P("d"), P("d")),
            out_specs=P("d"), check_rep=False,
        )(coords, coordsT, cb, ctb, fhi, flo)
        out = shard_map(
            mlp_call, mesh=mesh,
            in_specs=(P(),) * 13,
            out_specs=P(), check_rep=False,
        )(agg.reshape(B * N, IN_CH), *weights)
    else:
        agg = knn_call(coords, coordsT, cb, ctb, fhi, flo)
        out = mlp_call(agg.reshape(B * N, IN_CH), *weights)

    return out.reshape(B, N, OUT_CH)
